# baseline (speedup 1.0000x reference)
import functools

import jax
import jax.numpy as jnp
from jax import lax
from jax.experimental import pallas as pl
from jax.experimental.pallas import tpu as pltpu
from jax.experimental.pallas import tpu_sc as plsc

B = 16384
D = 16
NC = 2
NS = 16
NW = NC * NS
ROWS_PER_W = B // NW
CHUNK = 128
NCHUNK = ROWS_PER_W // CHUNK


def _sc_gather_body(uid, iid, uemb, iemb, gu, gi, idx_v, rows_v, sem):
    wid = lax.axis_index("s") * NC + lax.axis_index("c")
    base = wid * ROWS_PER_W
    for j in range(NCHUNK):
        pltpu.sync_copy(uid.at[pl.ds(base + j * CHUNK, CHUNK)], idx_v.at[j])
        pltpu.sync_copy(iid.at[pl.ds(base + j * CHUNK, CHUNK)],
                        idx_v.at[NCHUNK + j])
    copies = []
    for j in range(NCHUNK):
        copies.append(pltpu.async_copy(uemb.at[idx_v.at[j]], rows_v.at[j], sem))
    for j in range(NCHUNK):
        copies.append(
            pltpu.async_copy(iemb.at[idx_v.at[NCHUNK + j]],
                             rows_v.at[NCHUNK + j], sem))
    for c in copies:
        c.wait()
    for j in range(NCHUNK):
        pltpu.sync_copy(rows_v.at[j], gu.at[pl.ds(base + j * CHUNK, CHUNK)])
        pltpu.sync_copy(rows_v.at[NCHUNK + j],
                        gi.at[pl.ds(base + j * CHUNK, CHUNK)])


@jax.jit
def _sc_gather(uid, iid, uemb, iemb):
    mesh = plsc.VectorSubcoreMesh(core_axis_name="c", subcore_axis_name="s")
    return pl.kernel(
        _sc_gather_body,
        out_type=(
            jax.ShapeDtypeStruct((B, D), jnp.float32),
            jax.ShapeDtypeStruct((B, D), jnp.float32),
        ),
        mesh=mesh,
        scratch_types=[
            pltpu.VMEM((2 * NCHUNK, CHUNK), jnp.int32),
            pltpu.VMEM((2 * NCHUNK, CHUNK, D), jnp.float32),
            pltpu.SemaphoreType.DMA,
        ],
        compiler_params=pltpu.CompilerParams(use_tc_tiling_on_sc=False),
    )(uid, iid, uemb, iemb)


BLK = 2048


def _mlp_body(gu, gi, w1u, w1i, b1, w2, b2, w3, b3, out):
    h = jnp.dot(gu[...], w1u[...], preferred_element_type=jnp.float32)
    h = h + jnp.dot(gi[...], w1i[...], preferred_element_type=jnp.float32)
    h = jnp.maximum(h + b1[...], 0.0)
    h = jnp.maximum(
        jnp.dot(h, w2[...], preferred_element_type=jnp.float32) + b2[...], 0.0)
    out[...] = jnp.dot(h, w3[...], preferred_element_type=jnp.float32) + b3[...]


def _mlp(gu, gi, W1, b1, W2, b2, W3, b3):
    w1u = W1[:D, :]
    w1i = W1[D:, :]
    b1r = jnp.reshape(b1, (1, -1))
    b2r = jnp.reshape(b2, (1, -1))
    b3r = jnp.reshape(b3, (1, -1))
    grid = (B // BLK,)
    return pl.pallas_call(
        _mlp_body,
        grid=grid,
        in_specs=[
            pl.BlockSpec((BLK, D), lambda i: (i, 0)),
            pl.BlockSpec((BLK, D), lambda i: (i, 0)),
            pl.BlockSpec(w1u.shape, lambda i: (0, 0)),
            pl.BlockSpec(w1i.shape, lambda i: (0, 0)),
            pl.BlockSpec(b1r.shape, lambda i: (0, 0)),
            pl.BlockSpec(W2.shape, lambda i: (0, 0)),
            pl.BlockSpec(b2r.shape, lambda i: (0, 0)),
            pl.BlockSpec(W3.shape, lambda i: (0, 0)),
            pl.BlockSpec(b3r.shape, lambda i: (0, 0)),
        ],
        out_specs=pl.BlockSpec((BLK, 1), lambda i: (i, 0)),
        out_shape=jax.ShapeDtypeStruct((B, 1), jnp.float32),
    )(gu, gi, w1u, w1i, b1r, W2, b2r, W3, b3r)


def kernel(user_id, item_id, user_emb, item_emb, W1, b1, W2, b2, W3, b3):
    gu, gi = _sc_gather(user_id.astype(jnp.int32), item_id.astype(jnp.int32),
                        user_emb, item_emb)
    return _mlp(gu, gi, W1, b1, W2, b2, W3, b3)
